# Initial kernel scaffold; baseline (speedup 1.0000x reference)
import jax
import jax.numpy as jnp
from jax.experimental import pallas as pl


def kernel(corr_features, ref_mask):
    raise NotImplementedError("write your pallas kernel here")


# speedup vs baseline: 3.4951x; 3.4951x over previous
import functools

import numpy as np
import jax
import jax.numpy as jnp
from jax import lax
from jax.experimental import pallas as pl
from jax.experimental.pallas import tpu as pltpu
from jax.experimental.pallas import tpu_sc as plsc

KEEP = 32
B = 16
HW = 1024
NC, NS, L = 2, 16, 16
NW = NC * NS
ROWS_PER_W = B * HW // NW
CH = 32
NCHUNK = ROWS_PER_W // CH
NEG = np.float32(-3.0e38)


def _resize_weights(in_size: int = 512, out_size: int = 32) -> np.ndarray:
    inv_scale = np.float32(in_size / out_size)
    sample_f = (np.arange(out_size, dtype=np.float32) + np.float32(0.5)) * inv_scale - np.float32(0.5)
    x = np.abs(sample_f[None, :] - np.arange(in_size, dtype=np.float32)[:, None]) / inv_scale
    w = np.maximum(np.float32(0.0), np.float32(1.0) - x).astype(np.float32)
    w = w / w.sum(axis=0, keepdims=True)
    return w.astype(np.float32)


def _mask_tc(ref_mask, w):
    def body(m_ref, w_ref, o_ref):
        m = m_ref[0]
        ww = w_ref[...]
        t = lax.dot_general(ww, m, (((0,), (0,)), ((), ())),
                            precision=lax.Precision.HIGHEST)
        o = lax.dot_general(t, ww, (((1,), (0,)), ((), ())),
                            precision=lax.Precision.HIGHEST)
        o_ref[0] = (o > 0.5).astype(jnp.float32)

    return pl.pallas_call(
        body,
        grid=(B,),
        in_specs=[pl.BlockSpec((1, 512, 512), lambda i: (i, 0, 0)),
                  pl.BlockSpec((512, 32), lambda i: (0, 0))],
        out_specs=pl.BlockSpec((1, 32, 32), lambda i: (i, 0, 0)),
        out_shape=jax.ShapeDtypeStruct((B, 32, 32), jnp.float32),
    )(ref_mask, w)


def _sort16(v):
    return plsc.sort_key_val(v, v, descending=True)[0]


def _side_update(hi, lo, th, v):
    def merge(ops):
        hi, lo, v = ops
        sv = _sort16(v)
        lo2 = jnp.maximum(lo, lax.rev(sv, (0,)))
        h2 = jnp.maximum(hi, lo2)
        l2 = jnp.minimum(hi, lo2)
        hs = _sort16(h2)
        ls = _sort16(l2)
        nth = lax.gather(
            ls, jnp.full((L, 1), L - 1, jnp.int32),
            lax.GatherDimensionNumbers(offset_dims=(), collapsed_slice_dims=(0,),
                                       start_index_map=(0,)),
            (1,), mode=lax.GatherScatterMode.PROMISE_IN_BOUNDS)
        return hs, ls, nth

    def skip(ops):
        hi, lo, _ = ops
        return hi, lo, th

    return lax.cond(jnp.any(v > th), merge, skip, (hi, lo, v))


def _topk_sc(corr, mask):
    mesh = plsc.VectorSubcoreMesh(core_axis_name="c", subcore_axis_name="s")

    @functools.partial(
        pl.kernel,
        mesh=mesh,
        compiler_params=pltpu.CompilerParams(needs_layout_passes=False),
        out_type=jax.ShapeDtypeStruct((B, HW, 2 * KEEP), jnp.float32),
        scratch_types=[
            pltpu.VMEM((HW,), jnp.float32),
            pltpu.VMEM((2, CH, HW), jnp.float32),
            pltpu.VMEM((CH, 2 * KEEP), jnp.float32),
            pltpu.SemaphoreType.DMA,
            pltpu.SemaphoreType.DMA,
        ],
    )
    def k(corr_hbm, mask_hbm, out_hbm, mask_v, in_v, out_v, sem0, sem1):
        wid = lax.axis_index("c") * NS + lax.axis_index("s")
        b = wid // 2
        r0 = (wid % 2) * ROWS_PER_W
        sems = (sem0, sem1)

        pltpu.sync_copy(mask_hbm.at[b], mask_v)
        pltpu.async_copy(corr_hbm.at[b, pl.ds(r0, CH), :], in_v.at[0], sems[0])

        neg16 = jnp.full((L,), NEG, jnp.float32)

        def row_body(buf):
            def body(i, _):
                def vec_body(j, carry):
                    fh, fl, ft, bh, bl, bt = carry
                    v = in_v[buf, i, pl.ds(j * L, L)]
                    mm = mask_v[pl.ds(j * L, L)]
                    fv = v * mm
                    bv = v - fv
                    fh, fl, ft = _side_update(fh, fl, ft, fv)
                    bh, bl, bt = _side_update(bh, bl, bt, bv)
                    return fh, fl, ft, bh, bl, bt

                fh, fl, _, bh, bl, _ = lax.fori_loop(
                    0, HW // L, vec_body,
                    (neg16, neg16, neg16, neg16, neg16, neg16))
                out_v[i, pl.ds(0, L)] = bh
                out_v[i, pl.ds(L, L)] = bl
                out_v[i, pl.ds(2 * L, L)] = fh
                out_v[i, pl.ds(3 * L, L)] = fl
                return 0

            lax.fori_loop(0, CH, body, 0)

        for g in range(NCHUNK):
            buf = g % 2
            pltpu.make_async_copy(
                corr_hbm.at[b, pl.ds(r0 + g * CH, CH), :],
                in_v.at[buf], sems[buf]).wait()
            if g + 1 < NCHUNK:
                pltpu.async_copy(
                    corr_hbm.at[b, pl.ds(r0 + (g + 1) * CH, CH), :],
                    in_v.at[1 - buf], sems[1 - buf])
            row_body(buf)
            pltpu.sync_copy(out_v, out_hbm.at[b, pl.ds(r0 + g * CH, CH), :])

    return k(corr, mask)


def kernel(corr_features, ref_mask):
    batch, ch, cw, rh, rw = corr_features.shape
    corr = corr_features.reshape(batch, ch * cw, rh * rw)
    w = jnp.asarray(_resize_weights(ref_mask.shape[-1], rh))
    fg = _mask_tc(ref_mask.reshape(batch, ref_mask.shape[-2], ref_mask.shape[-1]), w)
    fg = fg.reshape(batch, rh * rw)
    out = _topk_sc(corr, fg)
    out = out.transpose(0, 2, 1).reshape(batch, 2 * KEEP, ch, cw)
    return out
